# baseline (speedup 1.0000x reference)
import jax
import jax.numpy as jnp
from jax import lax
from jax.experimental import pallas as pl
from jax.experimental.pallas import tpu as pltpu
from jax.experimental.pallas import tpu_sc as plsc

B = 16384
D = 32
L = 16
NC = 2
NS = 16
NW = NC * NS
BPW = B // NW
LANE = 128
NBLK = 7813
BLKW = 245
NB = 4
NCH = (BLKW + NB - 1) // NB
FRAME = 64
CAP = 784
NENT = NW * NCH * 2 * FRAME
SENT0 = 2 * B
NROWS = SENT0 + NENT


def _stage1(uid_hbm, iid_hbm, ut_hbm, it_hbm, vals_hbm, tidx_hbm,
            uid_v, iid_v, ub_v, ur_v, ib_v, ir_v, stage, cm_b, cm_r,
            cvals, cidx, sem_s):
    w = lax.axis_index("s") * NC + lax.axis_index("c")
    lanes = lax.iota(jnp.int32, L)

    pltpu.sync_copy(uid_hbm.at[:], uid_v)
    pltpu.sync_copy(iid_hbm.at[:], iid_v)

    blk0 = w * BLKW

    def filt(tt, ids_v, lb, lr):
        def body(j, cnt):
            idv = ids_v[pl.ds(j * L, L)]
            bv = j * L + lanes
            m = (idv >> 7) // BLKW == w
            nm = plsc.all_reduce_population_count(m)[0]
            plsc.store_compressed(lb.at[pl.ds(cnt, L)], bv, mask=m)
            plsc.store_compressed(lr.at[pl.ds(cnt, L)], idv, mask=m)
            return cnt + nm
        return lax.fori_loop(0, B // L, body, 0)

    ucnt = filt(0, uid_v, ub_v, ur_v)
    icnt = filt(1, iid_v, ib_v, ir_v)
    cnts = [ucnt, icnt]
    lbs = [ub_v, ib_v]
    lrs = [ur_v, ir_v]

    def fire(c, buf, tbl_ref, tt):
        cps = []
        for j in range(NB):
            bi = jnp.minimum(blk0 + c * NB + j, NBLK - 1)
            off = pl.multiple_of(bi * LANE, LANE)
            s = ((tt * 2 + buf) * NB + j) * D
            cps.append(pltpu.async_copy(
                tbl_ref.at[:, pl.ds(off, LANE)],
                stage.at[pl.ds(pl.multiple_of(s, D), D), :], sem_s))
        return cps

    def extract(c, buf, tt, cnt):
        c0 = blk0 + c * NB
        def scan(j, mcnt):
            rv = lrs[tt][pl.ds(j * L, L)]
            bv = lbs[tt][pl.ds(j * L, L)]
            blkv = rv >> 7
            m = (blkv >= c0) & (blkv < c0 + NB) & (j * L + lanes < cnt)
            nm = plsc.all_reduce_population_count(m)[0]
            plsc.store_compressed(cm_b.at[pl.ds(mcnt, L)], bv, mask=m)
            plsc.store_compressed(cm_r.at[pl.ds(mcnt, L)], rv, mask=m)
            return mcnt + nm
        mcnt = lax.fori_loop(0, (CAP + L - 1) // L, scan, 0)

        frame = (w * NCH + c) * 2 + tt
        for q in range(FRAME // L):
            cidx[pl.ds(q * L, L)] = SENT0 + frame * FRAME + q * L + lanes
        def wgrp(q):
            @pl.when(q * L < mcnt)
            def _():
                m = q * L + lanes < mcnt
                bv = cm_b[pl.ds(q * L, L)]
                rv = cm_r[pl.ds(q * L, L)]
                blkl = (rv >> 7) - c0
                rl = rv & (LANE - 1)
                plsc.store_scatter(cidx, [q * L + lanes], tt * B + bv, mask=m)
                srow = ((tt * 2 + buf) * NB + blkl) * D
                for d in range(D):
                    dv = jnp.full((L,), d, jnp.int32)
                    g = plsc.load_gather(stage, [srow + dv, rl], mask=m)
                    plsc.store_scatter(
                        cvals, [(q * L + lanes) * D + d], g, mask=m)
        for q in range(FRAME // L):
            wgrp(q)
        pltpu.sync_copy(cvals, vals_hbm.at[pl.ds(frame * FRAME * D, FRAME * D)])
        pltpu.sync_copy(cidx, tidx_hbm.at[pl.ds(frame * FRAME, FRAME)])

    for tt in range(2):
        tbl = ut_hbm if tt == 0 else it_hbm
        cps = fire(0, 0, tbl, tt)

        def chunk(c, _, tbl=tbl, tt=tt, cnt=cnts[tt]):
            @pl.when(c + 1 < NCH)
            def _():
                fire(c + 1, (c + 1) % 2, tbl, tt)
            for j in range(NB):
                pltpu.make_async_copy(
                    tbl.at[:, pl.ds(0, LANE)],
                    stage.at[pl.ds(0, D), :], sem_s).wait()
            extract(c, c % 2, tt, cnt)
            return 0

        lax.fori_loop(0, NCH, chunk, 0)


def _stage2(vals_hbm, tidx_hbm, rows_hbm, vals_v, idx_v, sem):
    w = lax.axis_index("s") * NC + lax.axis_index("c")
    nfr = NCH * 2

    def frame(f, _):
        fr = w * nfr + f
        pltpu.sync_copy(tidx_hbm.at[pl.ds(fr * FRAME, FRAME)], idx_v)
        pltpu.sync_copy(vals_hbm.at[pl.ds(fr * FRAME, FRAME), :], vals_v)
        pltpu.async_copy(vals_v, rows_hbm.at[idx_v], sem).wait()
        return 0

    lax.fori_loop(0, nfr, frame, 0)


def _stage3(rows_hbm, out_hbm, urows_v, irows_v, out_v):
    w = lax.axis_index("s") * NC + lax.axis_index("c")
    base = w * BPW
    pltpu.sync_copy(rows_hbm.at[pl.ds(base * D, BPW * D)], urows_v)
    pltpu.sync_copy(rows_hbm.at[pl.ds((B + base) * D, BPW * D)], irows_v)
    lanes = lax.iota(jnp.int32, L)

    def group(g, _):
        bvec = g * L + lanes
        acc = jnp.zeros((L,), jnp.float32)
        for d in range(D):
            dv = jnp.full((L,), d, jnp.int32)
            acc = acc + (plsc.load_gather(urows_v, [bvec * D + dv]) *
                         plsc.load_gather(irows_v, [bvec * D + dv]))
        out_v[pl.ds(g * L, L)] = acc
        return 0

    lax.fori_loop(0, BPW // L, group, 0)
    pltpu.sync_copy(out_v, out_hbm.at[pl.ds(base, BPW)])


def kernel(user_ids, item_ids, user_table, item_table):
    mesh = plsc.VectorSubcoreMesh(core_axis_name="c", subcore_axis_name="s")
    ut = user_table.T
    it = item_table.T

    k1 = pl.kernel(
        _stage1,
        mesh=mesh,
        compiler_params=pltpu.CompilerParams(
            use_tc_tiling_on_sc=True, needs_layout_passes=False),
        out_type=(jax.ShapeDtypeStruct((NENT * D,), jnp.float32),
                  jax.ShapeDtypeStruct((NENT,), jnp.int32)),
        scratch_types=[
            pltpu.VMEM((B,), jnp.int32),
            pltpu.VMEM((B,), jnp.int32),
            pltpu.VMEM((CAP + L,), jnp.int32),
            pltpu.VMEM((CAP + L,), jnp.int32),
            pltpu.VMEM((CAP + L,), jnp.int32),
            pltpu.VMEM((CAP + L,), jnp.int32),
            pltpu.VMEM((2 * 2 * NB * D, LANE), jnp.float32),
            pltpu.VMEM((FRAME + L,), jnp.int32),
            pltpu.VMEM((FRAME + L,), jnp.int32),
            pltpu.VMEM((FRAME * D,), jnp.float32),
            pltpu.VMEM((FRAME,), jnp.int32),
            pltpu.SemaphoreType.DMA,
        ],
    )
    vals, tidx = k1(user_ids.astype(jnp.int32), item_ids.astype(jnp.int32),
                    ut, it)

    k2 = pl.kernel(
        _stage2,
        mesh=mesh,
        compiler_params=pltpu.CompilerParams(
            use_tc_tiling_on_sc=False, needs_layout_passes=False),
        out_type=jax.ShapeDtypeStruct((NROWS, D), jnp.float32),
        scratch_types=[
            pltpu.VMEM((FRAME, D), jnp.float32),
            pltpu.VMEM((FRAME,), jnp.int32),
            pltpu.SemaphoreType.DMA,
        ],
    )
    rows = k2(vals.reshape(NENT, D), tidx)

    k3 = pl.kernel(
        _stage3,
        mesh=mesh,
        compiler_params=pltpu.CompilerParams(
            use_tc_tiling_on_sc=False, needs_layout_passes=False),
        out_type=jax.ShapeDtypeStruct((B,), jnp.float32),
        scratch_types=[
            pltpu.VMEM((BPW * D,), jnp.float32),
            pltpu.VMEM((BPW * D,), jnp.float32),
            pltpu.VMEM((BPW,), jnp.float32),
        ],
    )
    return k3(rows.reshape(NROWS * D))


# speedup vs baseline: 1.6119x; 1.6119x over previous
import jax
import jax.numpy as jnp
from jax import lax
from jax.experimental import pallas as pl
from jax.experimental.pallas import tpu as pltpu
from jax.experimental.pallas import tpu_sc as plsc

B = 16384
D = 32
L = 16
NC = 2
NS = 16
NW = NC * NS
BPW = B // NW
LANE = 128
NBLK = 7813
BLKW = 245
NB = 8
NCH = (BLKW + NB - 1) // NB
FRAME = 64
CAP = 784
NENT = NW * NCH * 2 * FRAME
SENT0 = 2 * B
NROWS = SENT0 + NENT


def _stage1(uid_hbm, iid_hbm, ut_hbm, it_hbm, vals_hbm, tidx_hbm,
            uid_v, iid_v, ub_v, ur_v, ib_v, ir_v, stage, cm_b, cm_r,
            cvals, cidx, sem_s):
    w = lax.axis_index("s") * NC + lax.axis_index("c")
    lanes = lax.iota(jnp.int32, L)

    pltpu.sync_copy(uid_hbm.at[:], uid_v)
    pltpu.sync_copy(iid_hbm.at[:], iid_v)

    blk0 = w * BLKW

    def filt_body(j, cnts):
        ucnt, icnt = cnts
        uv = uid_v[pl.ds(j * L, L)]
        iv = iid_v[pl.ds(j * L, L)]
        bv = j * L + lanes
        mu = (uv >> 7) // BLKW == w
        mi = (iv >> 7) // BLKW == w
        nu = plsc.all_reduce_population_count(mu)[0]
        ni = plsc.all_reduce_population_count(mi)[0]
        plsc.store_compressed(ub_v.at[pl.ds(ucnt, L)], bv, mask=mu)
        plsc.store_compressed(ur_v.at[pl.ds(ucnt, L)], uv, mask=mu)
        plsc.store_compressed(ib_v.at[pl.ds(icnt, L)], bv, mask=mi)
        plsc.store_compressed(ir_v.at[pl.ds(icnt, L)], iv, mask=mi)
        return ucnt + nu, icnt + ni

    ucnt, icnt = lax.fori_loop(0, B // L, filt_body, (0, 0))
    cnts = [ucnt, icnt]
    lbs = [ub_v, ib_v]
    lrs = [ur_v, ir_v]

    def fire(c, buf, tbl_ref):
        for j in range(NB):
            bi = jnp.minimum(blk0 + c * NB + j, NBLK - 1)
            off = pl.multiple_of(bi * LANE, LANE)
            s = (buf * NB + j) * D
            pltpu.async_copy(
                tbl_ref.at[:, pl.ds(off, LANE)],
                stage.at[pl.ds(pl.multiple_of(s, D), D), :], sem_s)

    def extract(c, buf, tt, cnt):
        c0 = blk0 + c * NB
        def scan(j, mcnt):
            rv = lrs[tt][pl.ds(j * L, L)]
            bv = lbs[tt][pl.ds(j * L, L)]
            blkv = rv >> 7
            m = (blkv >= c0) & (blkv < c0 + NB) & (j * L + lanes < cnt)
            nm = plsc.all_reduce_population_count(m)[0]
            plsc.store_compressed(cm_b.at[pl.ds(mcnt, L)], bv, mask=m)
            plsc.store_compressed(cm_r.at[pl.ds(mcnt, L)], rv, mask=m)
            return mcnt + nm
        mcnt = lax.fori_loop(0, (cnt + L - 1) // L, scan, 0)

        frame = (w * NCH + c) * 2 + tt
        for q in range(FRAME // L):
            cidx[pl.ds(q * L, L)] = SENT0 + frame * FRAME + q * L + lanes
        for q in range(FRAME // L):
            @pl.when(q * L < mcnt)
            def _(q=q):
                m = q * L + lanes < mcnt
                bv = cm_b[pl.ds(q * L, L)]
                rv = cm_r[pl.ds(q * L, L)]
                blkl = (rv >> 7) - c0
                rl = rv & (LANE - 1)
                plsc.store_scatter(cidx, [q * L + lanes], tt * B + bv, mask=m)
                srow = (buf * NB + blkl) * D
                for d in range(D):
                    dv = jnp.full((L,), d, jnp.int32)
                    g = plsc.load_gather(stage, [srow + dv, rl], mask=m)
                    plsc.store_scatter(
                        cvals, [(q * L + lanes) * D + d], g, mask=m)
        pltpu.sync_copy(cvals, vals_hbm.at[pl.ds(frame * FRAME * D, FRAME * D)])
        pltpu.sync_copy(cidx, tidx_hbm.at[pl.ds(frame * FRAME, FRAME)])

    for tt in range(2):
        tbl = ut_hbm if tt == 0 else it_hbm
        fire(0, 0, tbl)

        def chunk(c, _, tbl=tbl, tt=tt, cnt=cnts[tt]):
            @pl.when(c + 1 < NCH)
            def _():
                fire(c + 1, (c + 1) % 2, tbl)
            for j in range(NB):
                pltpu.make_async_copy(
                    tbl.at[:, pl.ds(0, LANE)],
                    stage.at[pl.ds(0, D), :], sem_s).wait()
            extract(c, c % 2, tt, cnt)
            return 0

        lax.fori_loop(0, NCH, chunk, 0)


def _stage2(vals_hbm, tidx_hbm, rows_hbm, vals_v, idx_v, sem):
    w = lax.axis_index("s") * NC + lax.axis_index("c")
    nit = NCH * 2 * FRAME // LANE

    def it(f, _):
        e0 = (w * NCH * 2 * FRAME) + f * LANE
        pltpu.sync_copy(tidx_hbm.at[pl.ds(e0, LANE)], idx_v)
        pltpu.sync_copy(vals_hbm.at[pl.ds(e0, LANE), :], vals_v)
        pltpu.async_copy(vals_v, rows_hbm.at[idx_v], sem).wait()
        return 0

    lax.fori_loop(0, nit, it, 0)


def _stage3(rows_hbm, out_hbm, urows_v, irows_v, out_v):
    w = lax.axis_index("s") * NC + lax.axis_index("c")
    base = w * BPW
    pltpu.sync_copy(rows_hbm.at[pl.ds(base * D, BPW * D)], urows_v)
    pltpu.sync_copy(rows_hbm.at[pl.ds((B + base) * D, BPW * D)], irows_v)
    lanes = lax.iota(jnp.int32, L)

    def group(g, _):
        bvec = g * L + lanes
        acc = jnp.zeros((L,), jnp.float32)
        for d in range(D):
            dv = jnp.full((L,), d, jnp.int32)
            acc = acc + (plsc.load_gather(urows_v, [bvec * D + dv]) *
                         plsc.load_gather(irows_v, [bvec * D + dv]))
        out_v[pl.ds(g * L, L)] = acc
        return 0

    lax.fori_loop(0, BPW // L, group, 0)
    pltpu.sync_copy(out_v, out_hbm.at[pl.ds(base, BPW)])


def kernel(user_ids, item_ids, user_table, item_table):
    mesh = plsc.VectorSubcoreMesh(core_axis_name="c", subcore_axis_name="s")
    ut = user_table.T
    it = item_table.T

    k1 = pl.kernel(
        _stage1,
        mesh=mesh,
        compiler_params=pltpu.CompilerParams(
            use_tc_tiling_on_sc=True, needs_layout_passes=False),
        out_type=(jax.ShapeDtypeStruct((NENT * D,), jnp.float32),
                  jax.ShapeDtypeStruct((NENT,), jnp.int32)),
        scratch_types=[
            pltpu.VMEM((B,), jnp.int32),
            pltpu.VMEM((B,), jnp.int32),
            pltpu.VMEM((CAP + L,), jnp.int32),
            pltpu.VMEM((CAP + L,), jnp.int32),
            pltpu.VMEM((CAP + L,), jnp.int32),
            pltpu.VMEM((CAP + L,), jnp.int32),
            pltpu.VMEM((2 * NB * D, LANE), jnp.float32),
            pltpu.VMEM((FRAME + L,), jnp.int32),
            pltpu.VMEM((FRAME + L,), jnp.int32),
            pltpu.VMEM((FRAME * D,), jnp.float32),
            pltpu.VMEM((FRAME,), jnp.int32),
            pltpu.SemaphoreType.DMA,
        ],
    )
    vals, tidx = k1(user_ids.astype(jnp.int32), item_ids.astype(jnp.int32),
                    ut, it)

    k2 = pl.kernel(
        _stage2,
        mesh=mesh,
        compiler_params=pltpu.CompilerParams(
            use_tc_tiling_on_sc=False, needs_layout_passes=False),
        out_type=jax.ShapeDtypeStruct((NROWS, D), jnp.float32),
        scratch_types=[
            pltpu.VMEM((LANE, D), jnp.float32),
            pltpu.VMEM((LANE,), jnp.int32),
            pltpu.SemaphoreType.DMA,
        ],
    )
    rows = k2(vals.reshape(NENT, D), tidx)

    k3 = pl.kernel(
        _stage3,
        mesh=mesh,
        compiler_params=pltpu.CompilerParams(
            use_tc_tiling_on_sc=False, needs_layout_passes=False),
        out_type=jax.ShapeDtypeStruct((B,), jnp.float32),
        scratch_types=[
            pltpu.VMEM((BPW * D,), jnp.float32),
            pltpu.VMEM((BPW * D,), jnp.float32),
            pltpu.VMEM((BPW,), jnp.float32),
        ],
    )
    return k3(rows.reshape(NROWS * D))


# speedup vs baseline: 1.8633x; 1.1560x over previous
import jax
import jax.numpy as jnp
from jax import lax
from jax.experimental import pallas as pl
from jax.experimental.pallas import tpu as pltpu
from jax.experimental.pallas import tpu_sc as plsc

B = 16384
D = 32
L = 16
NC = 2
NS = 16
NW = NC * NS
BPW = B // NW
LANE = 128
SLOTS = 8


def _sc_body(uid_hbm, iid_hbm, ut_hbm, it_hbm, out_hbm,
             uid_v, iid_v, ustage, istage, out_v, tmp_v, sem_u, sem_i):
    wid = lax.axis_index("s") * NC + lax.axis_index("c")
    base = wid * BPW

    pltpu.sync_copy(uid_hbm.at[pl.ds(base, BPW)], uid_v)
    pltpu.sync_copy(iid_hbm.at[pl.ds(base, BPW)], iid_v)

    lanes = lax.iota(jnp.int32, L)
    active_lo = lanes < SLOTS
    zeros = jnp.zeros((L,), jnp.float32)

    def group(g, _):
        uvec = uid_v[pl.ds(g * L, L)]
        ivec = iid_v[pl.ds(g * L, L)]
        url = uvec & (LANE - 1)
        irl = ivec & (LANE - 1)
        svec = lanes & (SLOTS - 1)
        halves = []
        for half in range(2):
            cps = []
            for k in range(SLOTS):
                p = half * SLOTS + k
                ru = pl.multiple_of((uvec[p] >> 7) * LANE, LANE)
                ri = pl.multiple_of((ivec[p] >> 7) * LANE, LANE)
                cps.append(pltpu.async_copy(
                    ut_hbm.at[:, pl.ds(ru, LANE)], ustage.at[k], sem_u))
                cps.append(pltpu.async_copy(
                    it_hbm.at[:, pl.ds(ri, LANE)], istage.at[k], sem_i))
            for cp in cps:
                cp.wait()
            if half == 0:
                rlu, rli = url, irl
            else:
                rlu = plsc.load_gather(uid_v, [g * L + SLOTS + svec]) & (LANE - 1)
                rli = plsc.load_gather(iid_v, [g * L + SLOTS + svec]) & (LANE - 1)
            acc = zeros
            for d in range(D):
                dvec = jnp.full((L,), d, jnp.int32)
                ug = plsc.load_gather(ustage, [svec, dvec, rlu])
                vg = plsc.load_gather(istage, [svec, dvec, rli])
                acc = acc + ug * vg
            halves.append(acc)
        tmp_v[...] = halves[1]
        shifted = plsc.load_gather(tmp_v, [svec])
        out_v[pl.ds(g * L, L)] = jnp.where(active_lo, halves[0], shifted)
        return 0

    lax.fori_loop(0, BPW // L, group, 0)
    pltpu.sync_copy(out_v, out_hbm.at[pl.ds(base, BPW)])


def kernel(user_ids, item_ids, user_table, item_table):
    ut = user_table.T
    it = item_table.T
    mesh = plsc.VectorSubcoreMesh(core_axis_name="c", subcore_axis_name="s")
    f = pl.kernel(
        _sc_body,
        mesh=mesh,
        compiler_params=pltpu.CompilerParams(
            use_tc_tiling_on_sc=True, needs_layout_passes=False),
        out_type=jax.ShapeDtypeStruct((B,), jnp.float32),
        scratch_types=[
            pltpu.VMEM((BPW,), jnp.int32),
            pltpu.VMEM((BPW,), jnp.int32),
            pltpu.VMEM((SLOTS, D, LANE), jnp.float32),
            pltpu.VMEM((SLOTS, D, LANE), jnp.float32),
            pltpu.VMEM((BPW,), jnp.float32),
            pltpu.VMEM((L,), jnp.float32),
            pltpu.SemaphoreType.DMA,
            pltpu.SemaphoreType.DMA,
        ],
    )
    return f(user_ids.astype(jnp.int32), item_ids.astype(jnp.int32), ut, it)
